# speedup vs baseline: 1.0441x; 1.0441x over previous
import functools

import jax
import jax.numpy as jnp
from jax import lax
from jax.experimental import pallas as pl
from jax.experimental.pallas import tpu as pltpu

_FEAT_C = 64
_OUT_C = 32
_POOL = 64


def _body(C, Hf, Wf, x_ref, pw_ref, m_ref, bp_ref, g_ref, bc_ref, o_ref):
    f32 = jnp.float32
    bf16 = jnp.bfloat16

    x = x_ref[...]
    y = x.reshape(C * Hf, _POOL, x.shape[-1]).sum(axis=1)

    xp = jnp.dot(y.astype(bf16), pw_ref[...],
                 preferred_element_type=f32)

    f_pre = jnp.dot(m_ref[...], xp.astype(bf16),
                    preferred_element_type=f32)
    feat = jnp.maximum(f_pre + bp_ref[...], 0.0).astype(bf16)

    wi = lax.broadcasted_iota(jnp.int32, (Wf, Wf), 0)
    wo = lax.broadcasted_iota(jnp.int32, (Wf, Wf), 1)
    s_m1 = (wi == wo - 1).astype(bf16)
    s_p1 = (wi == wo + 1).astype(bf16)
    f_m1 = jnp.dot(feat, s_m1, preferred_element_type=f32).astype(bf16)
    f_p1 = jnp.dot(feat, s_p1, preferred_element_type=f32).astype(bf16)

    acc = jnp.dot(g_ref[0], f_m1, preferred_element_type=f32)
    acc = acc + jnp.dot(g_ref[1], feat, preferred_element_type=f32)
    acc = acc + jnp.dot(g_ref[2], f_p1, preferred_element_type=f32)
    o_ref[...] = jnp.maximum(acc + bc_ref[...], 0.0)


def kernel(img, w_proj, b_proj, w_conv, b_conv):
    B, C, H, W = img.shape
    Hf, Wf = H // _POOL, W // _POOL
    Hc, Wc = Hf * _POOL, Wf * _POOL

    x2d = img[:, :, :Hc, :Wc].reshape(B, C * Hc, Wc).astype(jnp.float32)

    pw = ((jnp.arange(Wc)[:, None] // _POOL == jnp.arange(Wf)[None, :])
          .astype(jnp.float32) / _POOL).astype(jnp.bfloat16)

    wp = w_proj.astype(jnp.float32) / _POOL
    eye_h = jnp.eye(Hf, dtype=jnp.float32)
    M = (jnp.einsum('cd,hk->hdck', wp, eye_h)
         .reshape(Hf * _FEAT_C, C * Hf).astype(jnp.bfloat16))
    bp_col = jnp.tile(b_proj.astype(jnp.float32),
                      Hf).reshape(Hf * _FEAT_C, 1)

    wc = w_conv.astype(jnp.float32)
    dy = jnp.arange(Hf)[None, :] - jnp.arange(Hf)[:, None] + 1
    valid = ((dy >= 0) & (dy <= 2)).astype(jnp.float32)
    T = wc[jnp.clip(dy, 0, 2)] * valid[:, :, None, None, None]
    G = (jnp.transpose(T, (2, 4, 0, 1, 3))
         .reshape(3, _OUT_C * Hf, Hf * _FEAT_C).astype(jnp.bfloat16))
    bc_col = jnp.repeat(b_conv.astype(jnp.float32),
                        Hf).reshape(_OUT_C * Hf, 1)

    body = functools.partial(_body, C, Hf, Wf)

    out2d = pl.pallas_call(
        body,
        out_shape=jax.ShapeDtypeStruct((B, _OUT_C * Hf, Wf), jnp.float32),
        grid_spec=pltpu.PrefetchScalarGridSpec(
            num_scalar_prefetch=0,
            grid=(B,),
            in_specs=[
                pl.BlockSpec((None, C * Hc, Wc), lambda b: (b, 0, 0)),
                pl.BlockSpec(pw.shape, lambda b: (0, 0)),
                pl.BlockSpec(M.shape, lambda b: (0, 0)),
                pl.BlockSpec(bp_col.shape, lambda b: (0, 0)),
                pl.BlockSpec(G.shape, lambda b: (0, 0, 0)),
                pl.BlockSpec(bc_col.shape, lambda b: (0, 0)),
            ],
            out_specs=pl.BlockSpec((None, _OUT_C * Hf, Wf),
                                   lambda b: (b, 0, 0)),
        ),
        compiler_params=pltpu.CompilerParams(
            dimension_semantics=("parallel",)),
    )(x2d, pw, M, bp_col, G, bc_col)

    return out2d.reshape(B, _OUT_C, Hf, Wf)
